# baseline (speedup 1.0000x reference)
import functools

import jax
import jax.numpy as jnp
from jax.experimental import pallas as pl
from jax.experimental.pallas import tpu as pltpu

N = 10000
E = 640000
B = 256
HID = 512
BOND = 128
STEREO = 32
HEADS = 8
HD = HID // HEADS
LAYERS = 4
SHARED = 512
TASK = 256
NCLS = 3


def _ln(x, g, b, eps=1e-5):
    m = jnp.mean(x, axis=-1, keepdims=True)
    v = jnp.mean((x - m) ** 2, axis=-1, keepdims=True)
    return (x - m) * jax.lax.rsqrt(v + eps) * g + b


def _node_enc_body(x_ref, wbe0, bbe0, gbe, bbe, wbe1, bbe1,
                   wse0, bse0, gse, bse, wse1, bse1,
                   wg, bg, wf_b, wf_s, bf, gf, bff, o_ref):
    x = x_ref[...]
    b = jax.nn.relu(_ln(jnp.dot(x, wbe0[...], preferred_element_type=jnp.float32) + bbe0[...],
                        gbe[...], bbe[...]))
    b = jnp.dot(b, wbe1[...], preferred_element_type=jnp.float32) + bbe1[...]
    s = jnp.tanh(_ln(jnp.dot(x, wse0[...], preferred_element_type=jnp.float32) + bse0[...],
                     gse[...], bse[...]))
    s = jnp.dot(s, wse1[...], preferred_element_type=jnp.float32) + bse1[...]
    gate = jax.nn.sigmoid(jnp.dot(x, wg[...], preferred_element_type=jnp.float32) + bg[0])
    s = s * gate
    h = jnp.dot(b, wf_b[...], preferred_element_type=jnp.float32) \
        + jnp.dot(s, wf_s[...], preferred_element_type=jnp.float32) + bf[...]
    o_ref[...] = jax.nn.relu(_ln(h, gf[...], bff[...]))


def _node_encoder(x, p):
    w0 = jnp.zeros((86, HID), jnp.float32).at[:75].set(p["ne_be0"]["w"])
    ws0 = jnp.zeros((86, STEREO), jnp.float32).at[75:].set(p["ne_se0"]["w"])
    wg = jnp.zeros((86, 1), jnp.float32).at[75:].set(p["ne_gate"]["w"])
    wf = p["ne_fuse"]["w"]
    R = 1000
    grid = (N // R,)
    args = (
        x, w0, p["ne_be0"]["b"], p["ne_be_ln"]["g"], p["ne_be_ln"]["b"],
        p["ne_be1"]["w"], p["ne_be1"]["b"],
        ws0, p["ne_se0"]["b"], p["ne_se_ln"]["g"], p["ne_se_ln"]["b"],
        p["ne_se1"]["w"], p["ne_se1"]["b"],
        wg, p["ne_gate"]["b"],
        wf[:HID], wf[HID:], p["ne_fuse"]["b"],
        p["ne_fuse_ln"]["g"], p["ne_fuse_ln"]["b"],
    )
    full = lambda a: pl.BlockSpec(a.shape, lambda i: (0,) * a.ndim)
    in_specs = [pl.BlockSpec((R, 86), lambda i: (i, 0))] + [full(a) for a in args[1:]]
    return pl.pallas_call(
        _node_enc_body,
        grid=grid,
        in_specs=in_specs,
        out_specs=pl.BlockSpec((R, HID), lambda i: (i, 0)),
        out_shape=jax.ShapeDtypeStruct((N, HID), jnp.float32),
    )(*args)


def _edge_enc_body(ea_ref, wb0, bb0, gb, bb, ws0, bs0, gs, bs,
                   wf_b, wf_s, bf, o_ref):
    x = ea_ref[...]
    eb = jax.nn.relu(_ln(jnp.dot(x, wb0[...], preferred_element_type=jnp.float32) + bb0[...],
                         gb[...], bb[...]))
    es = jnp.tanh(_ln(jnp.dot(x, ws0[...], preferred_element_type=jnp.float32) + bs0[...],
                      gs[...], bs[...]))
    o_ref[...] = jnp.dot(eb, wf_b[...], preferred_element_type=jnp.float32) \
        + jnp.dot(es, wf_s[...], preferred_element_type=jnp.float32) + bf[...]


def _edge_encoder(edge_attr, p):
    wb0 = jnp.zeros((18, BOND), jnp.float32).at[:11].set(p["ee_b0"]["w"])
    ws0 = jnp.zeros((18, 16), jnp.float32).at[11:].set(p["ee_s0"]["w"])
    wf = p["ee_fuse"]["w"]
    R = 4000
    args = (
        edge_attr, wb0, p["ee_b0"]["b"], p["ee_b_ln"]["g"], p["ee_b_ln"]["b"],
        ws0, p["ee_s0"]["b"], p["ee_s_ln"]["g"], p["ee_s_ln"]["b"],
        wf[:BOND], wf[BOND:], p["ee_fuse"]["b"],
    )
    full = lambda a: pl.BlockSpec(a.shape, lambda i: (0,) * a.ndim)
    in_specs = [pl.BlockSpec((R, 18), lambda i: (i, 0))] + [full(a) for a in args[1:]]
    return pl.pallas_call(
        _edge_enc_body,
        grid=(E // R,),
        in_specs=in_specs,
        out_specs=pl.BlockSpec((R, BOND), lambda i: (i, 0)),
        out_shape=jax.ShapeDtypeStruct((E, BOND), jnp.float32),
    )(*args)


def kernel(x, edge_index, edge_attr, batch, params):
    p = params
    h = _node_encoder(x, p)
    ea = _edge_encoder(edge_attr, p)
    src = edge_index[0]
    dst = edge_index[1]
    for g in p["gat"]:
        res = h
        xl = (h @ g["lin_l"]["w"] + g["lin_l"]["b"]).reshape(N, HEADS, HD)
        xr = (h @ g["lin_r"]["w"] + g["lin_r"]["b"]).reshape(N, HEADS, HD)
        ee = (ea @ g["we"]).reshape(E, HEADS, HD)
        m = jax.nn.leaky_relu(xl[src] + xr[dst] + ee, 0.2)
        alpha = jnp.einsum("ehd,hd->eh", m, g["att"])
        amax = jax.ops.segment_max(alpha, dst, num_segments=N)
        amax = jnp.where(jnp.isfinite(amax), amax, 0.0)
        exp_a = jnp.exp(alpha - amax[dst])
        denom = jax.ops.segment_sum(exp_a, dst, num_segments=N)
        a = exp_a / (denom[dst] + 1e-16)
        msg = a[..., None] * xl[src]
        out = jax.ops.segment_sum(msg, dst, num_segments=N).reshape(N, HID) + g["bias"]
        out = jax.nn.relu(_ln(out, g["ln"]["g"], g["ln"]["b"]))
        h = out + res
    attn_in = jnp.tanh(h @ p["ro_a0"]["w"] + p["ro_a0"]["b"])
    attn = attn_in @ p["ro_a1"]["w"] + p["ro_a1"]["b"]
    amax = jax.ops.segment_max(attn, batch, num_segments=B)
    amax = jnp.where(jnp.isfinite(amax), amax, 0.0)
    exp_a = jnp.exp(attn - amax[batch])
    denom = jax.ops.segment_sum(exp_a, batch, num_segments=B)
    attn = exp_a / (denom[batch] + 1e-16)
    xt = (h @ p["ro_tr"]["w"] + p["ro_tr"]["b"]).reshape(N, HEADS, HD)
    graph_emb = jax.ops.segment_sum((attn[..., None] * xt).reshape(N, HID), batch, num_segments=B)
    shared = jax.nn.relu(_ln(graph_emb @ p["sh"]["w"] + p["sh"]["b"],
                             p["sh_ln"]["g"], p["sh_ln"]["b"]))
    logits = []
    for t in ["DAT", "NET", "SERT"]:
        hp = p["heads"][t]
        z = jax.nn.relu(_ln(shared @ hp["l0"]["w"] + hp["l0"]["b"],
                            hp["ln"]["g"], hp["ln"]["b"]))
        z = jax.nn.relu(z @ hp["l1"]["w"] + hp["l1"]["b"])
        logits.append(z @ hp["l2"]["w"] + hp["l2"]["b"])
    return jnp.stack(logits, axis=0)


# speedup vs baseline: 7.8994x; 7.8994x over previous
import functools

import jax
import jax.numpy as jnp
from jax import lax
from jax.experimental import pallas as pl
from jax.experimental.pallas import tpu as pltpu
from jax.experimental.pallas import tpu_sc as plsc

NC = 2
NS = 16
NW = NC * NS
_SC_MESH = dict(core_axis_name="c", subcore_axis_name="s")

N = 10000
E = 640000
B = 256
HID = 512
BOND = 128
STEREO = 32
HEADS = 8
HD = HID // HEADS
LAYERS = 4
SHARED = 512
TASK = 256
NCLS = 3


def _bt(x):
    return x.astype(jnp.bfloat16).astype(jnp.float32)


def _ln(x, g, b, eps=1e-5):
    m = jnp.mean(x, axis=-1, keepdims=True)
    v = jnp.mean((x - m) ** 2, axis=-1, keepdims=True)
    return (x - m) / jnp.sqrt(v + eps) * g + b


def _node_enc_body(x_ref, wbe0, bbe0, gbe, bbe, wbe1, bbe1,
                   wse0, bse0, gse, bse, wse1, bse1,
                   wg, bg, wf_b, wf_s, bf, gf, bff, o_ref):
    x = x_ref[...]
    b = jax.nn.relu(_ln(jnp.dot(_bt(x), _bt(wbe0[...]), preferred_element_type=jnp.float32) + bbe0[...],
                        gbe[...], bbe[...]))
    b = jnp.dot(_bt(b), _bt(wbe1[...]), preferred_element_type=jnp.float32) + bbe1[...]
    s = jnp.tanh(_ln(jnp.dot(_bt(x), _bt(wse0[...]), preferred_element_type=jnp.float32) + bse0[...],
                     gse[...], bse[...]))
    s = jnp.dot(_bt(s), _bt(wse1[...]), preferred_element_type=jnp.float32) + bse1[...]
    gate = jax.nn.sigmoid(jnp.dot(_bt(x), _bt(wg[...]), preferred_element_type=jnp.float32) + bg[0])
    s = s * gate
    h = jnp.dot(_bt(b), _bt(wf_b[...]), preferred_element_type=jnp.float32) \
        + jnp.dot(_bt(s), _bt(wf_s[...]), preferred_element_type=jnp.float32) + bf[...]
    o_ref[...] = jax.nn.relu(_ln(h, gf[...], bff[...]))


def _node_encoder(x, p):
    w0 = jnp.zeros((86, HID), jnp.float32).at[:75].set(p["ne_be0"]["w"])
    ws0 = jnp.zeros((86, STEREO), jnp.float32).at[75:].set(p["ne_se0"]["w"])
    wg = jnp.zeros((86, 1), jnp.float32).at[75:].set(p["ne_gate"]["w"])
    wf = p["ne_fuse"]["w"]
    R = 1000
    grid = (N // R,)
    args = (
        x, w0, p["ne_be0"]["b"], p["ne_be_ln"]["g"], p["ne_be_ln"]["b"],
        p["ne_be1"]["w"], p["ne_be1"]["b"],
        ws0, p["ne_se0"]["b"], p["ne_se_ln"]["g"], p["ne_se_ln"]["b"],
        p["ne_se1"]["w"], p["ne_se1"]["b"],
        wg, p["ne_gate"]["b"],
        wf[:HID], wf[HID:], p["ne_fuse"]["b"],
        p["ne_fuse_ln"]["g"], p["ne_fuse_ln"]["b"],
    )
    full = lambda a: pl.BlockSpec(a.shape, lambda i: (0,) * a.ndim)
    in_specs = [pl.BlockSpec((R, 86), lambda i: (i, 0))] + [full(a) for a in args[1:]]
    return pl.pallas_call(
        _node_enc_body,
        grid=grid,
        in_specs=in_specs,
        out_specs=pl.BlockSpec((R, HID), lambda i: (i, 0)),
        out_shape=jax.ShapeDtypeStruct((N, HID), jnp.float32),
    )(*args)


def _edge_enc_body(ea_ref, wb0, bb0, gb, bb, ws0, bs0, gs, bs,
                   wf_b, wf_s, bf, o_ref):
    x = ea_ref[...]
    eb = jax.nn.relu(_ln(jnp.dot(_bt(x), _bt(wb0[...]), preferred_element_type=jnp.float32) + bb0[...],
                         gb[...], bb[...]))
    es = jnp.tanh(_ln(jnp.dot(_bt(x), _bt(ws0[...]), preferred_element_type=jnp.float32) + bs0[...],
                      gs[...], bs[...]))
    o_ref[...] = jnp.dot(_bt(eb), _bt(wf_b[...]), preferred_element_type=jnp.float32) \
        + jnp.dot(_bt(es), _bt(wf_s[...]), preferred_element_type=jnp.float32) + bf[...]


def _edge_encoder(edge_attr, p):
    wb0 = jnp.zeros((18, BOND), jnp.float32).at[:11].set(p["ee_b0"]["w"])
    ws0 = jnp.zeros((18, 16), jnp.float32).at[11:].set(p["ee_s0"]["w"])
    wf = p["ee_fuse"]["w"]
    R = 4000
    args = (
        edge_attr, wb0, p["ee_b0"]["b"], p["ee_b_ln"]["g"], p["ee_b_ln"]["b"],
        ws0, p["ee_s0"]["b"], p["ee_s_ln"]["g"], p["ee_s_ln"]["b"],
        wf[:BOND], wf[BOND:], p["ee_fuse"]["b"],
    )
    full = lambda a: pl.BlockSpec(a.shape, lambda i: (0,) * a.ndim)
    in_specs = [pl.BlockSpec((R, 18), lambda i: (i, 0))] + [full(a) for a in args[1:]]
    return pl.pallas_call(
        _edge_enc_body,
        grid=(E // R,),
        in_specs=in_specs,
        out_specs=pl.BlockSpec((R, BOND), lambda i: (i, 0)),
        out_shape=jax.ShapeDtypeStruct((E, BOND), jnp.float32),
    )(*args)


def _sc_permute_rows(table, idx, D, CB):
    n = idx.shape[0]
    per_w = n // NW
    n_b = per_w // CB

    @functools.partial(
        pl.kernel,
        out_type=jax.ShapeDtypeStruct((n, D), table.dtype),
        mesh=plsc.VectorSubcoreMesh(**_SC_MESH),
        scratch_types=[
            pltpu.VMEM((CB,), jnp.int32),
            pltpu.VMEM((CB, D), table.dtype),
            pltpu.SemaphoreType.DMA,
        ],
    )
    def k(table_hbm, idx_hbm, out_hbm, idx_v, buf, sem):
        wid = lax.axis_index("s") * NC + lax.axis_index("c")

        @pl.loop(0, n_b)
        def _(b):
            base = wid * per_w + b * CB
            pltpu.sync_copy(idx_hbm.at[pl.ds(base, CB)], idx_v)
            pltpu.async_copy(table_hbm.at[idx_v], buf, sem).wait()
            pltpu.sync_copy(buf, out_hbm.at[pl.ds(base, CB)])

    return k(table, idx)


def _sc_gather_add(xl, srcs, xr, dsts, CB=80):
    per_w = E // NW
    n_b = per_w // CB

    @functools.partial(
        pl.kernel,
        out_type=jax.ShapeDtypeStruct((E, HID), jnp.float32),
        mesh=plsc.VectorSubcoreMesh(**_SC_MESH),
        scratch_types=[
            pltpu.VMEM((CB,), jnp.int32),
            pltpu.VMEM((CB,), jnp.int32),
            pltpu.VMEM((CB, HID), jnp.float32),
            pltpu.SemaphoreType.DMA,
        ],
    )
    def k(xl_hbm, srcs_hbm, xr_hbm, dsts_hbm, out_hbm, si_v, di_v, buf, sem):
        wid = lax.axis_index("s") * NC + lax.axis_index("c")

        @pl.loop(0, n_b)
        def _(b):
            base = wid * per_w + b * CB
            pltpu.sync_copy(srcs_hbm.at[pl.ds(base, CB)], si_v)
            pltpu.sync_copy(dsts_hbm.at[pl.ds(base, CB)], di_v)
            pltpu.async_copy(xl_hbm.at[si_v], buf, sem).wait()
            pltpu.async_copy(xr_hbm.at[di_v], buf, sem, add=True).wait()
            pltpu.sync_copy(buf, out_hbm.at[pl.ds(base, CB)])

    return k(xl, srcs, xr, dsts)


def _vshift(v, idx):
    return lax.gather(
        v, idx[:, None],
        dimension_numbers=lax.GatherDimensionNumbers(
            offset_dims=(), collapsed_slice_dims=(0,), start_index_map=(0,)),
        slice_sizes=(1,),
        mode=lax.GatherScatterMode.PROMISE_IN_BOUNDS)


def _sc_segment_max(alpha, dsts):
    per_w = E // NW
    CBC = 160
    n_b = per_w // CBC
    HNP = 81920
    SL = HNP // NS

    @functools.partial(
        pl.kernel,
        out_type=jax.ShapeDtypeStruct((NW, HNP), jnp.float32),
        mesh=plsc.VectorSubcoreMesh(**_SC_MESH),
        compiler_params=pltpu.CompilerParams(needs_layout_passes=False),
        scratch_types=[
            pltpu.VMEM((HNP,), jnp.float32),
            pltpu.VMEM((CBC,), jnp.int32),
            pltpu.VMEM((CBC * HEADS,), jnp.float32),
        ],
    )
    def k(alpha_hbm, dst_hbm, out_hbm, tab, dst_v, alpha_v):
        cid = lax.axis_index("c")
        sid = lax.axis_index("s")
        wid = sid * NC + cid
        iota = lax.iota(jnp.int32, 16)

        @pl.loop(0, HNP // 16)
        def _(i):
            tab[pl.ds(i * 16, 16)] = jnp.full((16,), -3.0e38, jnp.float32)

        @pl.loop(0, n_b)
        def _(b):
            base = wid * per_w + b * CBC
            pltpu.sync_copy(dst_hbm.at[pl.ds(base, CBC)], dst_v)
            pltpu.sync_copy(alpha_hbm.at[pl.ds(base * HEADS, CBC * HEADS)],
                            alpha_v)

            @pl.loop(0, CBC // 16)
            def _(g):
                e0 = g * 16
                k16 = dst_v[pl.ds(e0, 16)]
                nxt = _vshift(k16, jnp.minimum(iota + 1, 15))
                islast = (k16 != nxt) | (iota == 15)
                for h in range(HEADS):
                    a = plsc.load_gather(alpha_v,
                                         [(e0 + iota) * HEADS + h])
                    for s in (1, 2, 4, 8):
                        pidx = jnp.maximum(iota - s, 0)
                        ks = _vshift(k16, pidx)
                        vs = _vshift(a, pidx)
                        a = jnp.where((iota >= s) & (ks == k16),
                                      jnp.maximum(a, vs), a)
                    ti = k16 + h * N
                    old = plsc.load_gather(tab, [ti])
                    plsc.store_scatter(tab, [ti], jnp.maximum(old, a),
                                       mask=islast)

        pltpu.sync_copy(tab, out_hbm.at[wid])

    parts = k(alpha.reshape(-1), dsts)

    CS = HNP // NW

    @functools.partial(
        pl.kernel,
        out_type=jax.ShapeDtypeStruct((HNP,), jnp.float32),
        mesh=plsc.VectorSubcoreMesh(**_SC_MESH),
        compiler_params=pltpu.CompilerParams(needs_layout_passes=False),
        scratch_types=[
            pltpu.VMEM((CS,), jnp.float32),
            pltpu.VMEM((CS,), jnp.float32),
        ],
    )
    def combine(parts_hbm, out_hbm, acc, tmp):
        wid = lax.axis_index("s") * NC + lax.axis_index("c")
        pltpu.sync_copy(parts_hbm.at[0, pl.ds(wid * CS, CS)], acc)
        for j in range(1, NW):
            pltpu.sync_copy(parts_hbm.at[j, pl.ds(wid * CS, CS)], tmp)

            @pl.loop(0, CS // 16)
            def _(i):
                sl_ = pl.ds(i * 16, 16)
                acc[sl_] = jnp.maximum(acc[sl_], tmp[sl_])

        pltpu.sync_copy(acc, out_hbm.at[pl.ds(wid * CS, CS)])

    return combine(parts)


def _sc_gather_amax(amax_c, dsts):
    per_w = E // NW
    CBC = 160
    n_b = per_w // CBC
    HNP = 81920

    @functools.partial(
        pl.kernel,
        out_type=jax.ShapeDtypeStruct((E * HEADS,), jnp.float32),
        mesh=plsc.VectorSubcoreMesh(**_SC_MESH),
        compiler_params=pltpu.CompilerParams(needs_layout_passes=False),
        scratch_types=[
            pltpu.VMEM((HNP,), jnp.float32),
            pltpu.VMEM((CBC,), jnp.int32),
            pltpu.VMEM((CBC * HEADS,), jnp.float32),
        ],
    )
    def k(amax_hbm, dst_hbm, out_hbm, tabv, dst_v, ambuf):
        wid = lax.axis_index("s") * NC + lax.axis_index("c")
        iota = lax.iota(jnp.int32, 16)
        pltpu.sync_copy(amax_hbm, tabv)

        @pl.loop(0, n_b)
        def _(b):
            base = wid * per_w + b * CBC
            pltpu.sync_copy(dst_hbm.at[pl.ds(base, CBC)], dst_v)

            @pl.loop(0, CBC // 16)
            def _(g):
                e0 = g * 16
                k16 = dst_v[pl.ds(e0, 16)]
                for h in range(HEADS):
                    v = plsc.load_gather(tabv, [k16 + h * N])
                    plsc.store_scatter(ambuf, [(e0 + iota) * HEADS + h], v)

            pltpu.sync_copy(
                ambuf, out_hbm.at[pl.ds(base * HEADS, CBC * HEADS)])

    return k(amax_c, dsts).reshape(E, HEADS)


def _exp_body(al_ref, am_ref, o_ref):
    o_ref[...] = jnp.exp(al_ref[...] - am_ref[...])


def _tc_exp(alpha, amg):
    T = 8000
    return pl.pallas_call(
        _exp_body,
        grid=(E // T,),
        in_specs=[pl.BlockSpec((T, HEADS), lambda i: (i, 0)),
                  pl.BlockSpec((T, HEADS), lambda i: (i, 0))],
        out_specs=pl.BlockSpec((T, HEADS), lambda i: (i, 0)),
        out_shape=jax.ShapeDtypeStruct((E, HEADS), jnp.float32),
    )(alpha, amg)


def _sc_message_pass(xlp, expa, srcs, dsts, eb):
    CB = 80
    NP = 10240
    RT = NP // NW
    DS = 384

    @functools.partial(
        pl.kernel,
        out_type=(jax.ShapeDtypeStruct((HEADS // 2, NP, 2 * HD), jnp.float32),
                  jax.ShapeDtypeStruct((HEADS, NW * DS), jnp.float32)),
        mesh=plsc.VectorSubcoreMesh(**_SC_MESH),
        compiler_params=pltpu.CompilerParams(needs_layout_passes=False),
        scratch_types=[
            pltpu.VMEM((RT + 8, 2 * HD), jnp.float32),
            pltpu.VMEM((DS,), jnp.float32),
            pltpu.VMEM((DS,), jnp.float32),
            pltpu.VMEM((CB,), jnp.int32),
            pltpu.VMEM((CB,), jnp.int32),
            pltpu.VMEM((CB,), jnp.int32),
            pltpu.VMEM((CB,), jnp.int32),
            pltpu.VMEM((CB * HEADS,), jnp.float32),
            pltpu.VMEM((CB,), jnp.float32),
            pltpu.VMEM((CB,), jnp.float32),
            pltpu.VMEM((CB, 2 * HD), jnp.float32),
            pltpu.VMEM((48,), jnp.int32),
            pltpu.SemaphoreType.DMA,
        ],
    )
    def k(xlp_hbm, expa_hbm, srcs_hbm, dsts_hbm, eb_hbm,
          out_hbm, den_hbm, tab, dta0, dta1, dst_v, src_v,
          idx_v, ld_v, albuf, ex0, ex1, rowbuf, ebv, sem):
        cid = lax.axis_index("c")
        sid = lax.axis_index("s")
        wid = sid * NC + cid
        iota = lax.iota(jnp.int32, 16)

        pltpu.sync_copy(eb_hbm, ebv)
        g16 = plsc.load_gather(ebv, [wid + jnp.minimum(iota, 1)])
        lo = g16[0]
        hi = g16[1]
        base0 = (lo // 8) * 8
        nb = jnp.maximum(0, (hi - base0 + CB - 1) // CB)
        n0 = wid * RT

        for hp in range(HEADS // 2):
            h0 = 2 * hp

            @pl.loop(0, RT + 8)
            def _(r):
                for c in range(8):
                    tab[r, pl.ds(c * 16, 16)] = jnp.zeros((16,), jnp.float32)

            @pl.loop(0, DS // 16)
            def _(i):
                z = jnp.zeros((16,), jnp.float32)
                dta0[pl.ds(i * 16, 16)] = z
                dta1[pl.ds(i * 16, 16)] = z

            @pl.loop(0, nb)
            def _(b):
                ub = base0 + b * CB
                base = jnp.minimum(ub, E - CB)
                vlo = jnp.maximum(lo, ub)
                pltpu.sync_copy(dsts_hbm.at[pl.ds(base, CB)], dst_v)
                pltpu.sync_copy(srcs_hbm.at[pl.ds(base, CB)], src_v)
                pltpu.sync_copy(
                    expa_hbm.at[pl.ds(base * HEADS, CB * HEADS)], albuf)

                @pl.loop(0, CB // 16)
                def _(g):
                    e0 = g * 16
                    k16 = dst_v[pl.ds(e0, 16)]
                    s16 = src_v[pl.ds(e0, 16)]
                    gi = base + e0 + iota
                    valid = (gi >= vlo) & (gi < hi)
                    ld = jnp.where(valid, k16 - n0, RT)
                    ldc = jnp.minimum(ld, RT - 1)
                    for j, (dta, exv) in enumerate(
                            ((dta0, ex0), (dta1, ex1))):
                        e_ = plsc.load_gather(
                            albuf, [(e0 + iota) * HEADS + h0 + j])
                        e_ = jnp.where(valid, e_, 0.0)
                        plsc.addupdate_scatter(dta, [ldc], e_, mask=valid)
                        exv[pl.ds(e0, 16)] = e_
                    idx_v[pl.ds(e0, 16)] = s16 * (HEADS // 2) + hp
                    ld_v[pl.ds(e0, 16)] = ld

                pltpu.async_copy(xlp_hbm.at[idx_v], rowbuf, sem).wait()

                @pl.loop(0, CB // 16)
                def _(g2):
                    e0 = g2 * 16
                    x0 = ex0[pl.ds(e0, 16)]
                    x1 = ex1[pl.ds(e0, 16)]
                    lds = ld_v[pl.ds(e0, 16)]
                    for i in range(16):
                        r = lds[i]
                        s0 = x0[i]
                        s1 = x1[i]
                        for c in range(8):
                            sl_ = pl.ds(c * 16, 16)
                            s_ = s0 if c < 4 else s1
                            tab[r, sl_] = (tab[r, sl_]
                                           + rowbuf[e0 + i, sl_] * s_)

            pltpu.sync_copy(tab.at[pl.ds(0, RT)],
                            out_hbm.at[hp, pl.ds(n0, RT)])
            pltpu.sync_copy(dta0, den_hbm.at[h0, pl.ds(wid * DS, DS)])
            pltpu.sync_copy(dta1, den_hbm.at[h0 + 1, pl.ds(wid * DS, DS)])

    return k(xlp, expa.reshape(-1), srcs, dsts, eb)


def _alpha_body(gsum_ref, ea_ref, we_ref, attf_ref, sel_ref, o_ref):
    ee = jnp.dot(_bt(ea_ref[...]), _bt(we_ref[...]), preferred_element_type=jnp.float32)
    m = jax.nn.leaky_relu(gsum_ref[...] + ee, 0.2)
    o_ref[...] = jnp.dot(_bt(m) * _bt(attf_ref[...]), sel_ref[...],
                         preferred_element_type=jnp.float32)


def _tc_alpha(gsum, ea_s, we, att):
    T = 2000
    attf = att.reshape(1, HID)
    sel = (jnp.arange(HID)[:, None] // HD == jnp.arange(HEADS)[None, :]
           ).astype(jnp.float32)
    full = lambda a: pl.BlockSpec(a.shape, lambda i: (0,) * a.ndim)
    return pl.pallas_call(
        _alpha_body,
        grid=(E // T,),
        in_specs=[pl.BlockSpec((T, HID), lambda i: (i, 0)),
                  pl.BlockSpec((T, BOND), lambda i: (i, 0)),
                  full(we), full(attf), full(sel)],
        out_specs=pl.BlockSpec((T, HEADS), lambda i: (i, 0)),
        out_shape=jax.ShapeDtypeStruct((E, HEADS), jnp.float32),
    )(gsum, ea_s, we, attf, sel)


def _proj_body(h_ref, wl, bl, wr, br, xl_ref, xr_ref):
    hv = h_ref[...]
    xl_ref[...] = jnp.dot(_bt(hv), _bt(wl[...]), preferred_element_type=jnp.float32) + bl[...]
    xr_ref[...] = jnp.dot(_bt(hv), _bt(wr[...]), preferred_element_type=jnp.float32) + br[...]


def _tc_proj(h, g):
    R = 1000
    full = lambda a: pl.BlockSpec(a.shape, lambda i: (0,) * a.ndim)
    args = (h, g["lin_l"]["w"], g["lin_l"]["b"], g["lin_r"]["w"], g["lin_r"]["b"])
    return pl.pallas_call(
        _proj_body,
        grid=(N // R,),
        in_specs=[pl.BlockSpec((R, HID), lambda i: (i, 0))] + [full(a) for a in args[1:]],
        out_specs=[pl.BlockSpec((R, HID), lambda i: (i, 0))] * 2,
        out_shape=[jax.ShapeDtypeStruct((N, HID), jnp.float32)] * 2,
    )(*args)


def _fin_body(m_ref, d_ref, res_ref, bias, lng, lnb, sel, o_ref):
    acc = jnp.concatenate([m_ref[i] for i in range(HEADS // 2)], axis=-1)
    den = lax.dot_general(d_ref[...], sel[...], (((1,), (0,)), ((), ())),
                          preferred_element_type=jnp.float32)
    out = acc / (den + 1e-16) + bias[...]
    out = jax.nn.relu(_ln(out, lng[...], lnb[...]))
    o_ref[...] = out + res_ref[...]


def _tc_finalize(msg, den_h, res, g):
    R = 1000
    sel = (jnp.arange(HEADS)[:, None] == jnp.arange(HID)[None, :] // HD
           ).astype(jnp.float32)
    full = lambda a: pl.BlockSpec(a.shape, lambda i: (0,) * a.ndim)
    args = (msg, den_h, res, g["bias"], g["ln"]["g"], g["ln"]["b"], sel)
    return pl.pallas_call(
        _fin_body,
        grid=(N // R,),
        in_specs=[pl.BlockSpec((HEADS // 2, R, 2 * HD), lambda i: (0, i, 0)),
                  pl.BlockSpec((R, HEADS), lambda i: (i, 0)),
                  pl.BlockSpec((R, HID), lambda i: (i, 0)),
                  full(args[3]), full(args[4]), full(args[5]), full(sel)],
        out_specs=pl.BlockSpec((R, HID), lambda i: (i, 0)),
        out_shape=jax.ShapeDtypeStruct((N, HID), jnp.float32),
    )(*args)


def _ro1_body(h_ref, batch_ref, a0w, a0b, a1w, a1b, am_ref):
    i = pl.program_id(0)

    @pl.when(i == 0)
    def _():
        am_ref[...] = jnp.full_like(am_ref, -3.0e38)

    attn = jnp.tanh(jnp.dot(_bt(h_ref[...]), _bt(a0w[...]),
                            preferred_element_type=jnp.float32) + a0b[...])
    attn = jnp.dot(_bt(attn), _bt(a1w[...]), preferred_element_type=jnp.float32) + a1b[...]
    bt = batch_ref[0, 0]
    onehot = bt[None, :] == lax.broadcasted_iota(jnp.int32, (B, 1), 0)
    cols = []
    for hh in range(HEADS):
        masked = jnp.where(onehot, attn[:, hh][None, :], -3.0e38)
        cols.append(masked.max(axis=1, keepdims=True))
    am_ref[...] = jnp.maximum(am_ref[...], jnp.concatenate(cols, axis=1))


def _tc_readout1(h, batch2, p):
    R = 1000
    full = lambda a: pl.BlockSpec(a.shape, lambda i: (0,) * a.ndim)
    args = (h, batch2, p["ro_a0"]["w"], p["ro_a0"]["b"],
            p["ro_a1"]["w"], p["ro_a1"]["b"])
    return pl.pallas_call(
        _ro1_body,
        grid=(N // R,),
        in_specs=[pl.BlockSpec((R, HID), lambda i: (i, 0)),
                  pl.BlockSpec((1, 1, R), lambda i: (i, 0, 0))]
        + [full(a) for a in args[2:]],
        out_specs=pl.BlockSpec((B, HEADS), lambda i: (0, 0)),
        out_shape=jax.ShapeDtypeStruct((B, HEADS), jnp.float32),
    )(*args)


def _ro2_body(h_ref, batch_ref, am_ref, a0w, a0b, a1w, a1b, trw, trb, sel,
              pool_ref, den_ref):
    i = pl.program_id(0)

    @pl.when(i == 0)
    def _():
        pool_ref[...] = jnp.zeros_like(pool_ref)
        den_ref[...] = jnp.zeros_like(den_ref)

    hv = h_ref[...]
    attn = jnp.tanh(jnp.dot(_bt(hv), _bt(a0w[...]),
                            preferred_element_type=jnp.float32) + a0b[...])
    attn = jnp.dot(_bt(attn), _bt(a1w[...]), preferred_element_type=jnp.float32) + a1b[...]
    bt = batch_ref[0, 0]
    onehot = (bt[None, :] == lax.broadcasted_iota(jnp.int32, (B, 1), 0)
              ).astype(jnp.float32)
    amax = jnp.where(am_ref[...] < -1e37, 0.0, am_ref[...])
    am_rows = lax.dot_general(onehot, amax, (((0,), (0,)), ((), ())),
                              preferred_element_type=jnp.float32)
    expa = jnp.exp(attn - am_rows)
    xt = jnp.dot(_bt(hv), _bt(trw[...]), preferred_element_type=jnp.float32) + trb[...]
    expe = jnp.dot(expa, sel[...], preferred_element_type=jnp.float32)
    pool_ref[...] += jnp.dot(onehot, expe * xt,
                             preferred_element_type=jnp.float32)
    den_ref[...] += jnp.dot(onehot, expa, preferred_element_type=jnp.float32)


def _tc_readout2(h, batch2, amax_g, p):
    R = 1000
    sel = (jnp.arange(HEADS)[:, None] == jnp.arange(HID)[None, :] // HD
           ).astype(jnp.float32)
    full = lambda a: pl.BlockSpec(a.shape, lambda i: (0,) * a.ndim)
    args = (h, batch2, amax_g, p["ro_a0"]["w"], p["ro_a0"]["b"],
            p["ro_a1"]["w"], p["ro_a1"]["b"], p["ro_tr"]["w"], p["ro_tr"]["b"],
            sel)
    return pl.pallas_call(
        _ro2_body,
        grid=(N // R,),
        in_specs=[pl.BlockSpec((R, HID), lambda i: (i, 0)),
                  pl.BlockSpec((1, 1, R), lambda i: (i, 0, 0))]
        + [full(a) for a in args[2:]],
        out_specs=[pl.BlockSpec((B, HID), lambda i: (0, 0)),
                   pl.BlockSpec((B, HEADS), lambda i: (0, 0))],
        out_shape=[jax.ShapeDtypeStruct((B, HID), jnp.float32),
                   jax.ShapeDtypeStruct((B, HEADS), jnp.float32)],
    )(*args)


def _head_body(pool_ref, den_ref, sel, shw, shb, shg, shbb, *hw):
    den = lax.dot_general(den_ref[...], sel[...], (((1,), (0,)), ((), ())),
                          preferred_element_type=jnp.float32)
    ge = pool_ref[...] / (den + 1e-16)
    sh = jax.nn.relu(_ln(jnp.dot(_bt(ge), _bt(shw[...]),
                                 preferred_element_type=jnp.float32) + shb[...],
                         shg[...], shbb[...]))
    outs = []
    for t in range(3):
        l0w, l0b, lng, lnb, l1w, l1b, l2w, l2b = hw[t * 8:(t + 1) * 8]
        z = jax.nn.relu(_ln(jnp.dot(_bt(sh), _bt(l0w[...]),
                                    preferred_element_type=jnp.float32) + l0b[...],
                            lng[...], lnb[...]))
        z = jax.nn.relu(jnp.dot(_bt(z), _bt(l1w[...]),
                                preferred_element_type=jnp.float32) + l1b[...])
        outs.append(jnp.dot(_bt(z), _bt(l2w[...]),
                            preferred_element_type=jnp.float32) + l2b[...])
    hw[-1][...] = jnp.stack(outs, axis=0)


def _tc_heads(pool, den, p):
    sel = (jnp.arange(HEADS)[:, None] == jnp.arange(HID)[None, :] // HD
           ).astype(jnp.float32)
    args = [pool, den, sel, p["sh"]["w"], p["sh"]["b"],
            p["sh_ln"]["g"], p["sh_ln"]["b"]]
    for t in ["DAT", "NET", "SERT"]:
        hp = p["heads"][t]
        args += [hp["l0"]["w"], hp["l0"]["b"], hp["ln"]["g"], hp["ln"]["b"],
                 hp["l1"]["w"], hp["l1"]["b"], hp["l2"]["w"], hp["l2"]["b"]]
    full = lambda a: pl.BlockSpec(a.shape, lambda i: (0,) * a.ndim)
    return pl.pallas_call(
        _head_body,
        grid=(1,),
        in_specs=[full(a) for a in args],
        out_specs=pl.BlockSpec((3, B, NCLS), lambda i: (0, 0, 0)),
        out_shape=jax.ShapeDtypeStruct((3, B, NCLS), jnp.float32),
    )(*args)


def kernel(x, edge_index, edge_attr, batch, params):
    p = params
    h = _node_encoder(x, p)
    ea = _edge_encoder(edge_attr, p)
    order = jnp.argsort(edge_index[1])
    srcs = edge_index[0][order]
    dsts = edge_index[1][order]
    ea_s = _sc_permute_rows(ea, order, BOND, 400)
    eb = jnp.zeros((48,), jnp.int32).at[:33].set(
        jnp.searchsorted(dsts, jnp.arange(33) * 320).astype(jnp.int32))
    for g in p["gat"]:
        xl, xr = _tc_proj(h, g)
        gsum = _sc_gather_add(xl, srcs, xr, dsts)
        alpha = _tc_alpha(gsum, ea_s, g["we"], g["att"])
        amax_c = _sc_segment_max(alpha, dsts)
        amg = _sc_gather_amax(amax_c, dsts)
        expa = _tc_exp(alpha, amg)
        xlp = xl.reshape(N * HEADS // 2, 2 * HD)
        msg, den = _sc_message_pass(xlp, expa, srcs, dsts, eb)
        den_h = den.reshape(HEADS, NW, 384)[:, :, :320].reshape(HEADS, -1).T
        h = _tc_finalize(msg, den_h, h, g)
    batch2 = batch.reshape(N // 1000, 1, 1000)
    amax_g = _tc_readout1(h, batch2, p)
    pool, deng = _tc_readout2(h, batch2, amax_g, p)
    return _tc_heads(pool, deng, p)
